# baseline (speedup 1.0000x reference)
import functools

import jax
import jax.numpy as jnp
import numpy as np
from jax import lax
from jax.experimental import pallas as pl
from jax.experimental.pallas import tpu as pltpu
from jax.experimental.pallas import tpu_sc as plsc

H = 128
EPF = 32
D = 448
HEADS = 4
NLAYERS = 4
LCODE = 23
MAXD = 2 * LCODE
NSTAB = 1058
NE = 4096
REMD = 4
RENC = 64
DEMB = 128
HD = D // 4
HDP = 128
NP = 1152
ECH = 1024
NCH = NE // ECH
BQ = 1024


def _ln(x, g, b):
    mu = jnp.mean(x, axis=-1, keepdims=True)
    var = jnp.mean((x - mu) ** 2, axis=-1, keepdims=True)
    return (x - mu) / jnp.sqrt(var + 1e-5) * g + b


def _dotT(x, w):
    return jax.lax.dot_general(x, w, (((1,), (1,)), ((), ())),
                               preferred_element_type=jnp.float32)


def _dot(a, b):
    return jax.lax.dot_general(a, b, (((1,), (0,)), ((), ())),
                               preferred_element_type=jnp.float32)


def _dotc0(a, b):
    return jax.lax.dot_general(a, b, (((0,), (0,)), ((), ())),
                               preferred_element_type=jnp.float32)


def _split(x):
    hi = x.astype(jnp.bfloat16)
    lo = (x - hi.astype(jnp.float32)).astype(jnp.bfloat16)
    return hi, lo


def _osel(o_bf, x):
    hi, lo = _split(x)
    d = (((1,), (0,)), ((), ()))
    return (jax.lax.dot_general(o_bf, hi, d, preferred_element_type=jnp.float32)
            + jax.lax.dot_general(o_bf, lo, d, preferred_element_type=jnp.float32))


def _oselc0(o_bf, x):
    hi, lo = _split(x)
    d = (((0,), (0,)), ((), ()))
    return (jax.lax.dot_general(o_bf, hi, d, preferred_element_type=jnp.float32)
            + jax.lax.dot_general(o_bf, lo, d, preferred_element_type=jnp.float32))


def _full(shape):
    return pl.BlockSpec(shape, lambda *_: tuple(0 for _ in shape))


def _node_embed_body(x_ref, spm_ref, emb_ref,
                     tw, tb, tg, tbb,
                     c1w, c1b, c2w, c2b, cg, cb,
                     d1w, d1b, d2w, d2b, dg, db,
                     p1w, p1b, p2w, p2b, pg, pb,
                     eg, eb, prw, prb, png, pnb,
                     out_ref):
    xv = x_ref[...]
    te = _ln(_dotT(xv[:, 0:2], tw[...]) + tb[...], tg[...], tbb[...])
    ce = _dotT(jax.nn.relu(_dotT(xv[:, 2:4], c1w[...]) + c1b[...]), c2w[...]) + c2b[...]
    ce = _ln(ce, cg[...], cb[...])
    de = _dotT(jax.nn.relu(_dotT(xv[:, 4:5], d1w[...]) + d1b[...]), d2w[...]) + d2b[...]
    de = _ln(de, dg[...], db[...])
    pe = _dotT(jax.nn.relu(_dotT(xv[:, 5:13], p1w[...]) + p1b[...]), p2w[...]) + p2b[...]
    pe = _ln(pe, pg[...], pb[...])
    feats = jnp.concatenate([te, ce, de, pe], axis=1)
    learn = _ln(emb_ref[...], eg[...], eb[...])
    embeds = jnp.concatenate([feats, learn], axis=1) * spm_ref[...]
    out_ref[...] = _ln(_dotT(embeds, prw[...]) + prb[...], png[...], pnb[...])


def _gnn_layer_body(h_ref, dst_ref, src_ref,
                    qw, qb, kw, kb, vw, vb,
                    sw, sb, bw, n1g, n1b, n2g, n2b,
                    f1w, f1b, f2w, f2b,
                    out_ref):
    h_in = h_ref[...]
    hn = _ln(h_in, n1g[...], n1b[...])
    q = _dotT(hn, qw[...]) + qb[...]
    k = _dotT(hn, kw[...]) + kb[...]
    v = _dotT(hn, vw[...]) + vb[...]

    iota_n = jax.lax.broadcasted_iota(jnp.int32, (ECH, NP), 1)
    iota_h = jax.lax.broadcasted_iota(jnp.int32, (HEADS * H, HEADS), 0)
    iota_hh = jax.lax.broadcasted_iota(jnp.int32, (HEADS * H, HEADS), 1)
    ones_blk = (iota_h // H == iota_hh).astype(jnp.float32)

    scale = 1.0 / np.sqrt(H)
    denom = jnp.zeros((NP, HEADS), jnp.float32)
    aggu = jnp.zeros((NP, HEADS * H), jnp.float32)
    for c in range(NCH):
        dst_c = dst_ref[c * ECH:(c + 1) * ECH, :]
        src_c = src_ref[c * ECH:(c + 1) * ECH, :]
        od = (iota_n == dst_c).astype(jnp.float32)
        os_ = (iota_n == src_c).astype(jnp.float32)
        qd = _dot(od, q)
        ks = _dot(os_, k)
        vs = _dot(os_, v)
        alpha = _dot(qd * ks, ones_blk) * scale
        ea = jnp.exp(alpha)
        eexp = jax.lax.dot_general(ea, ones_blk, (((1,), (1,)), ((), ())),
                                   preferred_element_type=jnp.float32)
        denom = denom + _dotc0(od, ea)
        aggu = aggu + _dotc0(od, vs * eexp)

    dexp = jax.lax.dot_general(denom, ones_blk, (((1,), (1,)), ((), ())),
                               preferred_element_type=jnp.float32)
    agg = aggu / (dexp + 1e-16)
    out = (agg[:, 0:H] + agg[:, H:2 * H] + agg[:, 2 * H:3 * H]
           + agg[:, 3 * H:4 * H]) * (1.0 / HEADS)
    skip = _dotT(hn, sw[...]) + sb[...]
    bcat = jnp.concatenate([out, skip, out - skip], axis=1)
    beta = jax.nn.sigmoid(_dotT(bcat, bw[...]))
    hatt = beta * skip + (1.0 - beta) * out
    h1 = hatt + h_in
    h2 = _ln(h1, n2g[...], n2b[...])
    g_in = _dotT(h2, f1w[...]) + f1b[...]
    gelu = g_in * 0.5 * (1.0 + jax.lax.erf(g_in * (1.0 / np.sqrt(2.0))))
    ff = _dotT(gelu, f2w[...]) + f2b[...]
    out_ref[...] = h1 + ff


def _sc_gather_rows(table, idx, b_total, dcols):
    info = plsc.get_sparse_core_info()
    nc, ns = info.num_cores, info.num_subcores
    nw = nc * ns
    bpw = b_total // nw
    mesh = plsc.VectorSubcoreMesh(core_axis_name="c", subcore_axis_name="s")

    @functools.partial(
        pl.kernel, mesh=mesh,
        out_type=jax.ShapeDtypeStruct((b_total, dcols), jnp.float32),
        scratch_types=[
            pltpu.VMEM((bpw,), jnp.int32),
            pltpu.VMEM((bpw, dcols), jnp.float32),
            pltpu.SemaphoreType.DMA,
        ],
    )
    def k(table_hbm, idx_hbm, out_hbm, idx_v, rows_v, sem):
        wid = lax.axis_index("s") * nc + lax.axis_index("c")
        base = wid * bpw
        pltpu.sync_copy(idx_hbm.at[pl.ds(base, bpw)], idx_v)
        pltpu.async_copy(table_hbm.at[idx_v], rows_v, sem).wait()
        pltpu.sync_copy(rows_v, out_hbm.at[pl.ds(base, bpw)])

    return k(table, idx)


def _assemble_body(hs_ref, hd_ref, eattr_ref, dtab_ref,
                   dg, db, r1w, r1b, r1g, r1b2, r2w, r2b, r2g, r2b2,
                   pg, pb,
                   out_ref):
    iota_d = jax.lax.broadcasted_iota(jnp.int32, (ECH, 128), 1)
    for c in range(NCH):
        rows = slice(c * ECH, (c + 1) * ECH)
        ea_c = eattr_ref[rows, :]
        hs = hs_ref[rows, :]
        hd = hd_ref[rows, :]
        ids = jnp.clip(ea_c[:, 0:1].astype(jnp.int32), 0, MAXD)
        demb = _dot((iota_d == ids).astype(jnp.float32), dtab_ref[...])
        demb = _ln(demb, dg[...], db[...])
        rem = ea_c[:, 1:5]
        rem = _ln(jax.nn.relu(_dotT(rem, r1w[...]) + r1b[...]), r1g[...], r1b2[...])
        rem = _ln(jax.nn.relu(_dotT(rem, r2w[...]) + r2b[...]), r2g[...], r2b2[...])
        ei = jnp.concatenate([hs, hd, demb, rem], axis=1)
        out_ref[rows, :] = _ln(ei, pg[...], pb[...])


def _tproj_body(t_ref, qw, qb, kw, kb, vw, vb, q_ref, k_ref, v_ref):
    t = t_ref[...]
    q_ref[...] = _dotT(t, qw[...]) + qb[...]
    k_ref[...] = _dotT(t, kw[...]) + kb[...]
    v_ref[...] = _dotT(t, vw[...]) + vb[...]


def _tattn_body(q_ref, k_ref, v_ref, t_ref,
                ow, ob, n1g, n1b, f1w, f1b, f2w, f2b, n2g, n2b,
                out_ref):
    scale = 1.0 / np.sqrt(HD)
    oh = []
    for hh in range(4):
        sl = slice(hh * HDP, (hh + 1) * HDP)
        s = jax.lax.dot_general(q_ref[:, sl] * scale, k_ref[:, sl],
                                (((1,), (1,)), ((), ())),
                                preferred_element_type=jnp.float32)
        m = jnp.max(s, axis=1, keepdims=True)
        p = jnp.exp(s - m)
        l = jnp.sum(p, axis=1, keepdims=True)
        oh.append(_dot(p / l, v_ref[:, sl]))
    o = jnp.concatenate(oh, axis=1)
    t = t_ref[...]
    t1 = _ln(t + _dotT(o, ow[...]) + ob[...], n1g[...], n1b[...])
    ff = _dotT(jax.nn.relu(_dotT(t1, f1w[...]) + f1b[...]), f2w[...]) + f2b[...]
    out_ref[...] = _ln(t1 + ff, n2g[...], n2b[...])


def _logits_body(t_ref, lw, lb, out_ref):
    out_ref[...] = _dotT(t_ref[...], lw[...]) + lb[...]


def _pc(body, out_shapes, args, grid=None, in_specs=None, out_specs=None):
    if grid is None:
        in_specs = [_full(a.shape) for a in args]
        if isinstance(out_shapes, tuple):
            out_specs = tuple(_full(s.shape) for s in out_shapes)
        else:
            out_specs = _full(out_shapes.shape)
    return pl.pallas_call(body, grid=grid or (1,), in_specs=in_specs,
                          out_specs=out_specs, out_shape=out_shapes)(*args)


def _row(d):
    return lambda i: (i, 0)


def kernel(x, edge_attr, params, edge_index, syndrome):
    p = params
    f32 = jnp.float32

    x_p = jnp.pad(x, ((0, NP - NSTAB), (0, 3)))
    spm_p = jnp.pad(syndrome.astype(f32) * 2.0 - 1.0,
                    (0, NP - NSTAB)).reshape(NP, 1)
    emb_p = jnp.pad(p['stab_emb'], ((0, NP - NSTAB), (0, 0)))
    src = edge_index[0].reshape(NE, 1)
    dst = edge_index[1].reshape(NE, 1)
    eattr_p = jnp.pad(edge_attr, ((0, 0), (0, 3)))
    dtab_p = jnp.pad(p['dist_table'], ((0, 128 - (MAXD + 1)), (0, 0)))

    def r2(a):
        return a.reshape(1, -1)

    h = _pc(_node_embed_body, jax.ShapeDtypeStruct((NP, H), f32), (
        x_p, spm_p, emb_p,
        p['type_W'], r2(p['type_b']), r2(p['type_norm_g']), r2(p['type_norm_b']),
        p['coords1_W'], r2(p['coords1_b']), p['coords2_W'], r2(p['coords2_b']),
        r2(p['coords_norm_g']), r2(p['coords_norm_b']),
        p['dc1_W'], r2(p['dc1_b']), p['dc2_W'], r2(p['dc2_b']),
        r2(p['dc_norm_g']), r2(p['dc_norm_b']),
        p['pe1_W'], r2(p['pe1_b']), p['pe2_W'], r2(p['pe2_b']),
        r2(p['pe_norm_g']), r2(p['pe_norm_b']),
        r2(p['emb_norm_g']), r2(p['emb_norm_b']),
        p['pre_W'], r2(p['pre_b']), r2(p['pre_norm_g']), r2(p['pre_norm_b']),
    ))

    for i in range(NLAYERS):
        pf = 'l%d_' % i
        h = _pc(_gnn_layer_body, jax.ShapeDtypeStruct((NP, H), f32), (
            h, dst, src,
            p[pf + 'q_W'], r2(p[pf + 'q_b']),
            p[pf + 'k_W'], r2(p[pf + 'k_b']),
            p[pf + 'v_W'], r2(p[pf + 'v_b']),
            p[pf + 'skip_W'], r2(p[pf + 'skip_b']), p[pf + 'beta_W'],
            r2(p[pf + 'norm1_g']), r2(p[pf + 'norm1_b']),
            r2(p[pf + 'norm2_g']), r2(p[pf + 'norm2_b']),
            p[pf + 'ffn1_W'], r2(p[pf + 'ffn1_b']),
            p[pf + 'ffn2_W'], r2(p[pf + 'ffn2_b']),
        ))

    sd_idx = jnp.concatenate([edge_index[0], edge_index[1]])
    hsd = _sc_gather_rows(h, sd_idx, 2 * NE, H)
    t = _pc(_assemble_body, jax.ShapeDtypeStruct((NE, D), f32), (
        hsd[:NE], hsd[NE:], eattr_p, dtab_p,
        r2(p['dist_norm_g']), r2(p['dist_norm_b']),
        p['rem1_W'], r2(p['rem1_b']), r2(p['rem_ln1_g']), r2(p['rem_ln1_b']),
        p['rem2_W'], r2(p['rem2_b']), r2(p['rem_ln2_g']), r2(p['rem_ln2_b']),
        r2(p['pred_norm_g']), r2(p['pred_norm_b']),
    ))

    for i in range(2):
        pf = 't%d_' % i
        wi = p[pf + 'inproj_W'].reshape(3, 4, HD, D)
        wi = jnp.pad(wi, ((0, 0), (0, 0), (0, HDP - HD), (0, 0)))
        bi = p[pf + 'inproj_b'].reshape(3, 4, HD)
        bi = jnp.pad(bi, ((0, 0), (0, 0), (0, HDP - HD)))
        wo = p[pf + 'outproj_W'].reshape(D, 4, HD)
        wo = jnp.pad(wo, ((0, 0), (0, 0), (0, HDP - HD))).reshape(D, 4 * HDP)

        qkv = pl.pallas_call(
            _tproj_body, grid=(NE // BQ,),
            in_specs=[pl.BlockSpec((BQ, D), _row(D))] + [_full(w.shape) for w in (
                wi[0].reshape(4 * HDP, D), r2(bi[0]),
                wi[1].reshape(4 * HDP, D), r2(bi[1]),
                wi[2].reshape(4 * HDP, D), r2(bi[2]))],
            out_specs=tuple(pl.BlockSpec((BQ, 4 * HDP), _row(4 * HDP)) for _ in range(3)),
            out_shape=tuple(jax.ShapeDtypeStruct((NE, 4 * HDP), f32) for _ in range(3)),
        )(t, wi[0].reshape(4 * HDP, D), r2(bi[0]),
          wi[1].reshape(4 * HDP, D), r2(bi[1]),
          wi[2].reshape(4 * HDP, D), r2(bi[2]))

        wargs = (wo, r2(p[pf + 'outproj_b']),
                 r2(p[pf + 'norm1_g']), r2(p[pf + 'norm1_b']),
                 p[pf + 'ff1_W'], r2(p[pf + 'ff1_b']),
                 p[pf + 'ff2_W'], r2(p[pf + 'ff2_b']),
                 r2(p[pf + 'norm2_g']), r2(p[pf + 'norm2_b']))
        t = pl.pallas_call(
            _tattn_body, grid=(NE // BQ,),
            in_specs=[pl.BlockSpec((BQ, 4 * HDP), _row(4 * HDP)),
                      _full((NE, 4 * HDP)), _full((NE, 4 * HDP)),
                      pl.BlockSpec((BQ, D), _row(D))]
                     + [_full(w.shape) for w in wargs],
            out_specs=pl.BlockSpec((BQ, D), _row(D)),
            out_shape=jax.ShapeDtypeStruct((NE, D), f32),
        )(qkv[0], qkv[1], qkv[2], t, *wargs)

    lw = jnp.pad(p['out_W'], ((0, 127), (0, 0)))
    lb = jnp.pad(p['out_b'], (0, 127)).reshape(1, 128)
    lg = _pc(_logits_body, jax.ShapeDtypeStruct((NE, 128), f32), (t, lw, lb))
    return lg[:, 0]


# speedup vs baseline: 1.2619x; 1.2619x over previous
import functools

import jax
import jax.numpy as jnp
import numpy as np
from jax.experimental import pallas as pl

H = 128
EPF = 32
D = 448
HEADS = 4
NLAYERS = 4
LCODE = 23
MAXD = 2 * LCODE
NSTAB = 1058
NE = 4096
REMD = 4
RENC = 64
DEMB = 128
HD = D // 4
HDP = 128
NP = 1152
ECH = 1024
NCH = NE // ECH
BQ = 1024


def _ln(x, g, b):
    mu = jnp.mean(x, axis=-1, keepdims=True)
    var = jnp.mean((x - mu) ** 2, axis=-1, keepdims=True)
    return (x - mu) / jnp.sqrt(var + 1e-5) * g + b


def _dotT(x, w):
    return jax.lax.dot_general(x, w, (((1,), (1,)), ((), ())),
                               preferred_element_type=jnp.float32)


def _dot(a, b):
    return jax.lax.dot_general(a, b, (((1,), (0,)), ((), ())),
                               preferred_element_type=jnp.float32)


def _dotc0(a, b):
    return jax.lax.dot_general(a, b, (((0,), (0,)), ((), ())),
                               preferred_element_type=jnp.float32)


def _split(x):
    hi = x.astype(jnp.bfloat16)
    lo = (x - hi.astype(jnp.float32)).astype(jnp.bfloat16)
    return hi, lo


def _osel(o_bf, x):
    hi, lo = _split(x)
    d = (((1,), (0,)), ((), ()))
    return (jax.lax.dot_general(o_bf, hi, d, preferred_element_type=jnp.float32)
            + jax.lax.dot_general(o_bf, lo, d, preferred_element_type=jnp.float32))


def _oselc0(o_bf, x):
    hi, lo = _split(x)
    d = (((0,), (0,)), ((), ()))
    return (jax.lax.dot_general(o_bf, hi, d, preferred_element_type=jnp.float32)
            + jax.lax.dot_general(o_bf, lo, d, preferred_element_type=jnp.float32))


def _full(shape):
    return pl.BlockSpec(shape, lambda *_: tuple(0 for _ in shape))


def _node_embed_body(x_ref, spm_ref, emb_ref,
                     tw, tb, tg, tbb,
                     c1w, c1b, c2w, c2b, cg, cb,
                     d1w, d1b, d2w, d2b, dg, db,
                     p1w, p1b, p2w, p2b, pg, pb,
                     eg, eb, prw, prb, png, pnb,
                     out_ref):
    xv = x_ref[...]
    te = _ln(_dotT(xv[:, 0:2], tw[...]) + tb[...], tg[...], tbb[...])
    ce = _dotT(jax.nn.relu(_dotT(xv[:, 2:4], c1w[...]) + c1b[...]), c2w[...]) + c2b[...]
    ce = _ln(ce, cg[...], cb[...])
    de = _dotT(jax.nn.relu(_dotT(xv[:, 4:5], d1w[...]) + d1b[...]), d2w[...]) + d2b[...]
    de = _ln(de, dg[...], db[...])
    pe = _dotT(jax.nn.relu(_dotT(xv[:, 5:13], p1w[...]) + p1b[...]), p2w[...]) + p2b[...]
    pe = _ln(pe, pg[...], pb[...])
    feats = jnp.concatenate([te, ce, de, pe], axis=1)
    learn = _ln(emb_ref[...], eg[...], eb[...])
    embeds = jnp.concatenate([feats, learn], axis=1) * spm_ref[...]
    out_ref[...] = _ln(_dotT(embeds, prw[...]) + prb[...], png[...], pnb[...])


def _gnn_layer_body(h_ref, dst_ref, src_ref,
                    qw, qb, kw, kb, vw, vb,
                    sw, sb, bw, n1g, n1b, n2g, n2b,
                    f1w, f1b, f2w, f2b,
                    out_ref):
    h_in = h_ref[...]
    hn = _ln(h_in, n1g[...], n1b[...])
    q = _dotT(hn, qw[...]) + qb[...]
    k = _dotT(hn, kw[...]) + kb[...]
    v = _dotT(hn, vw[...]) + vb[...]

    iota_n = jax.lax.broadcasted_iota(jnp.int32, (ECH, NP), 1)
    iota_h = jax.lax.broadcasted_iota(jnp.int32, (HEADS * H, HEADS), 0)
    iota_hh = jax.lax.broadcasted_iota(jnp.int32, (HEADS * H, HEADS), 1)
    ones_blk = (iota_h // H == iota_hh).astype(jnp.float32)

    scale = 1.0 / np.sqrt(H)
    denom = jnp.zeros((NP, HEADS), jnp.float32)
    aggu = jnp.zeros((NP, HEADS * H), jnp.float32)
    for c in range(NCH):
        dst_c = dst_ref[c * ECH:(c + 1) * ECH, :]
        src_c = src_ref[c * ECH:(c + 1) * ECH, :]
        od = (iota_n == dst_c).astype(jnp.float32)
        os_ = (iota_n == src_c).astype(jnp.float32)
        qd = _dot(od, q)
        ks = _dot(os_, k)
        vs = _dot(os_, v)
        alpha = _dot(qd * ks, ones_blk) * scale
        ea = jnp.exp(alpha)
        eexp = jax.lax.dot_general(ea, ones_blk, (((1,), (1,)), ((), ())),
                                   preferred_element_type=jnp.float32)
        denom = denom + _dotc0(od, ea)
        aggu = aggu + _dotc0(od, vs * eexp)

    dexp = jax.lax.dot_general(denom, ones_blk, (((1,), (1,)), ((), ())),
                               preferred_element_type=jnp.float32)
    agg = aggu / (dexp + 1e-16)
    out = (agg[:, 0:H] + agg[:, H:2 * H] + agg[:, 2 * H:3 * H]
           + agg[:, 3 * H:4 * H]) * (1.0 / HEADS)
    skip = _dotT(hn, sw[...]) + sb[...]
    bcat = jnp.concatenate([out, skip, out - skip], axis=1)
    beta = jax.nn.sigmoid(_dotT(bcat, bw[...]))
    hatt = beta * skip + (1.0 - beta) * out
    h1 = hatt + h_in
    h2 = _ln(h1, n2g[...], n2b[...])
    g_in = _dotT(h2, f1w[...]) + f1b[...]
    gelu = g_in * 0.5 * (1.0 + jax.lax.erf(g_in * (1.0 / np.sqrt(2.0))))
    ff = _dotT(gelu, f2w[...]) + f2b[...]
    out_ref[...] = h1 + ff


def _assemble_body(h_ref, src_ref, dst_ref, eattr_ref, dtab_ref,
                   dg, db, r1w, r1b, r1g, r1b2, r2w, r2b, r2g, r2b2,
                   pg, pb,
                   out_ref):
    h = h_ref[...]
    iota_n = jax.lax.broadcasted_iota(jnp.int32, (ECH, NP), 1)
    iota_d = jax.lax.broadcasted_iota(jnp.int32, (ECH, 128), 1)
    for c in range(NCH):
        rows = slice(c * ECH, (c + 1) * ECH)
        src_c = src_ref[rows, :]
        dst_c = dst_ref[rows, :]
        ea_c = eattr_ref[rows, :]
        hs = _dot((iota_n == src_c).astype(jnp.float32), h)
        hd = _dot((iota_n == dst_c).astype(jnp.float32), h)
        ids = jnp.clip(ea_c[:, 0:1].astype(jnp.int32), 0, MAXD)
        demb = _dot((iota_d == ids).astype(jnp.float32), dtab_ref[...])
        demb = _ln(demb, dg[...], db[...])
        rem = ea_c[:, 1:5]
        rem = _ln(jax.nn.relu(_dotT(rem, r1w[...]) + r1b[...]), r1g[...], r1b2[...])
        rem = _ln(jax.nn.relu(_dotT(rem, r2w[...]) + r2b[...]), r2g[...], r2b2[...])
        ei = jnp.concatenate([hs, hd, demb, rem], axis=1)
        out_ref[rows, :] = _ln(ei, pg[...], pb[...])


def _tproj_body(t_ref, qw, qb, kw, kb, vw, vb, q_ref, k_ref, v_ref):
    t = t_ref[...]
    q_ref[...] = _dotT(t, qw[...]) + qb[...]
    k_ref[...] = _dotT(t, kw[...]) + kb[...]
    v_ref[...] = _dotT(t, vw[...]) + vb[...]


def _tattn_body(q_ref, k_ref, v_ref, t_ref,
                ow, ob, n1g, n1b, f1w, f1b, f2w, f2b, n2g, n2b,
                out_ref):
    scale = 1.0 / np.sqrt(HD)
    oh = []
    for hh in range(4):
        sl = slice(hh * HDP, (hh + 1) * HDP)
        s = jax.lax.dot_general(q_ref[:, sl] * scale, k_ref[:, sl],
                                (((1,), (1,)), ((), ())),
                                preferred_element_type=jnp.float32)
        p = jnp.exp(s)
        l = jnp.sum(p, axis=1, keepdims=True)
        oh.append(_dot(p, v_ref[:, sl]) / l)
    o = jnp.concatenate(oh, axis=1)
    t = t_ref[...]
    t1 = _ln(t + _dotT(o, ow[...]) + ob[...], n1g[...], n1b[...])
    ff = _dotT(jax.nn.relu(_dotT(t1, f1w[...]) + f1b[...]), f2w[...]) + f2b[...]
    out_ref[...] = _ln(t1 + ff, n2g[...], n2b[...])


def _logits_body(t_ref, lw, lb, out_ref):
    out_ref[...] = _dotT(t_ref[...], lw[...]) + lb[...]


def _pc(body, out_shapes, args, grid=None, in_specs=None, out_specs=None):
    if grid is None:
        in_specs = [_full(a.shape) for a in args]
        if isinstance(out_shapes, tuple):
            out_specs = tuple(_full(s.shape) for s in out_shapes)
        else:
            out_specs = _full(out_shapes.shape)
    return pl.pallas_call(body, grid=grid or (1,), in_specs=in_specs,
                          out_specs=out_specs, out_shape=out_shapes)(*args)


def _row(d):
    return lambda i: (i, 0)


def kernel(x, edge_attr, params, edge_index, syndrome):
    p = params
    f32 = jnp.float32

    x_p = jnp.pad(x, ((0, NP - NSTAB), (0, 3)))
    spm_p = jnp.pad(syndrome.astype(f32) * 2.0 - 1.0,
                    (0, NP - NSTAB)).reshape(NP, 1)
    emb_p = jnp.pad(p['stab_emb'], ((0, NP - NSTAB), (0, 0)))
    src = edge_index[0].reshape(NE, 1)
    dst = edge_index[1].reshape(NE, 1)
    eattr_p = jnp.pad(edge_attr, ((0, 0), (0, 3)))
    dtab_p = jnp.pad(p['dist_table'], ((0, 128 - (MAXD + 1)), (0, 0)))

    def r2(a):
        return a.reshape(1, -1)

    h = _pc(_node_embed_body, jax.ShapeDtypeStruct((NP, H), f32), (
        x_p, spm_p, emb_p,
        p['type_W'], r2(p['type_b']), r2(p['type_norm_g']), r2(p['type_norm_b']),
        p['coords1_W'], r2(p['coords1_b']), p['coords2_W'], r2(p['coords2_b']),
        r2(p['coords_norm_g']), r2(p['coords_norm_b']),
        p['dc1_W'], r2(p['dc1_b']), p['dc2_W'], r2(p['dc2_b']),
        r2(p['dc_norm_g']), r2(p['dc_norm_b']),
        p['pe1_W'], r2(p['pe1_b']), p['pe2_W'], r2(p['pe2_b']),
        r2(p['pe_norm_g']), r2(p['pe_norm_b']),
        r2(p['emb_norm_g']), r2(p['emb_norm_b']),
        p['pre_W'], r2(p['pre_b']), r2(p['pre_norm_g']), r2(p['pre_norm_b']),
    ))

    for i in range(NLAYERS):
        pf = 'l%d_' % i
        h = _pc(_gnn_layer_body, jax.ShapeDtypeStruct((NP, H), f32), (
            h, dst, src,
            p[pf + 'q_W'], r2(p[pf + 'q_b']),
            p[pf + 'k_W'], r2(p[pf + 'k_b']),
            p[pf + 'v_W'], r2(p[pf + 'v_b']),
            p[pf + 'skip_W'], r2(p[pf + 'skip_b']), p[pf + 'beta_W'],
            r2(p[pf + 'norm1_g']), r2(p[pf + 'norm1_b']),
            r2(p[pf + 'norm2_g']), r2(p[pf + 'norm2_b']),
            p[pf + 'ffn1_W'], r2(p[pf + 'ffn1_b']),
            p[pf + 'ffn2_W'], r2(p[pf + 'ffn2_b']),
        ))

    t = _pc(_assemble_body, jax.ShapeDtypeStruct((NE, D), f32), (
        h, src, dst, eattr_p, dtab_p,
        r2(p['dist_norm_g']), r2(p['dist_norm_b']),
        p['rem1_W'], r2(p['rem1_b']), r2(p['rem_ln1_g']), r2(p['rem_ln1_b']),
        p['rem2_W'], r2(p['rem2_b']), r2(p['rem_ln2_g']), r2(p['rem_ln2_b']),
        r2(p['pred_norm_g']), r2(p['pred_norm_b']),
    ))

    for i in range(2):
        pf = 't%d_' % i
        wi = p[pf + 'inproj_W'].reshape(3, 4, HD, D)
        wi = jnp.pad(wi, ((0, 0), (0, 0), (0, HDP - HD), (0, 0)))
        bi = p[pf + 'inproj_b'].reshape(3, 4, HD)
        bi = jnp.pad(bi, ((0, 0), (0, 0), (0, HDP - HD)))
        wo = p[pf + 'outproj_W'].reshape(D, 4, HD)
        wo = jnp.pad(wo, ((0, 0), (0, 0), (0, HDP - HD))).reshape(D, 4 * HDP)

        qkv = pl.pallas_call(
            _tproj_body, grid=(NE // BQ,),
            in_specs=[pl.BlockSpec((BQ, D), _row(D))] + [_full(w.shape) for w in (
                wi[0].reshape(4 * HDP, D), r2(bi[0]),
                wi[1].reshape(4 * HDP, D), r2(bi[1]),
                wi[2].reshape(4 * HDP, D), r2(bi[2]))],
            out_specs=tuple(pl.BlockSpec((BQ, 4 * HDP), _row(4 * HDP)) for _ in range(3)),
            out_shape=tuple(jax.ShapeDtypeStruct((NE, 4 * HDP), f32) for _ in range(3)),
        )(t, wi[0].reshape(4 * HDP, D), r2(bi[0]),
          wi[1].reshape(4 * HDP, D), r2(bi[1]),
          wi[2].reshape(4 * HDP, D), r2(bi[2]))

        wargs = (wo, r2(p[pf + 'outproj_b']),
                 r2(p[pf + 'norm1_g']), r2(p[pf + 'norm1_b']),
                 p[pf + 'ff1_W'], r2(p[pf + 'ff1_b']),
                 p[pf + 'ff2_W'], r2(p[pf + 'ff2_b']),
                 r2(p[pf + 'norm2_g']), r2(p[pf + 'norm2_b']))
        t = pl.pallas_call(
            _tattn_body, grid=(NE // BQ,),
            in_specs=[pl.BlockSpec((BQ, 4 * HDP), _row(4 * HDP)),
                      _full((NE, 4 * HDP)), _full((NE, 4 * HDP)),
                      pl.BlockSpec((BQ, D), _row(D))]
                     + [_full(w.shape) for w in wargs],
            out_specs=pl.BlockSpec((BQ, D), _row(D)),
            out_shape=jax.ShapeDtypeStruct((NE, D), f32),
        )(qkv[0], qkv[1], qkv[2], t, *wargs)

    lw = jnp.pad(p['out_W'], ((0, 127), (0, 0)))
    lb = jnp.pad(p['out_b'], (0, 127)).reshape(1, 128)
    lg = _pc(_logits_body, jax.ShapeDtypeStruct((NE, 128), f32), (t, lw, lb))
    return lg[:, 0]
